# baseline (speedup 1.0000x reference)
import jax
import jax.numpy as jnp
from jax.experimental import pallas as pl

CLS = 1000
ROWS = 4096 * 26
BLOCK = 2048


def _onehot_block(x_ref, o_ref):
    xb = x_ref[...]
    iota = jax.lax.broadcasted_iota(jnp.int32, (BLOCK, CLS), 1)
    o_ref[...] = (xb == iota).astype(jnp.int32)


def kernel(x):
    xf = x.reshape(ROWS, 1)
    out = pl.pallas_call(
        _onehot_block,
        grid=(ROWS // BLOCK,),
        in_specs=[pl.BlockSpec((BLOCK, 1), lambda i: (i, 0))],
        out_specs=pl.BlockSpec((BLOCK, CLS), lambda i: (i, 0)),
        out_shape=jax.ShapeDtypeStruct((ROWS, CLS), jnp.int32),
    )(xf)
    return out.reshape(x.shape[0], x.shape[1], CLS)


# speedup vs baseline: 1.4505x; 1.4505x over previous
import jax
import jax.numpy as jnp
from jax.experimental import pallas as pl

CLS = 1000
B = 128


def _onehot_block(x_ref, o_ref):
    xb = x_ref[...]
    iota = jax.lax.broadcasted_iota(jnp.int32, (B, 26, CLS), 2)
    o_ref[...] = (xb[:, :, None] == iota).astype(jnp.int32)


def kernel(x):
    n, k = x.shape
    return pl.pallas_call(
        _onehot_block,
        grid=(n // B,),
        in_specs=[pl.BlockSpec((B, k), lambda i: (i, 0))],
        out_specs=pl.BlockSpec((B, k, CLS), lambda i: (i, 0, 0)),
        out_shape=jax.ShapeDtypeStruct((n, k, CLS), jnp.int32),
    )(x)
